# baseline (speedup 1.0000x reference)
import functools

import jax
import jax.numpy as jnp
from jax import lax
from jax.experimental import pallas as pl
from jax.experimental.pallas import tpu as pltpu
from jax.experimental.pallas import tpu_sc as plsc

_B, _T, _D, _H, _E, _K = 2, 2048, 2048, 2048, 8, 2
_N = _B * _T
_P = _N * _K
_BR = 512
_BH = 512
_RP = _P + _E * _BR - ((_P + _E * _BR) % _BR)
_NT = _RP // _BR
_NH = _H // _BH
_TG = 512

_NC, _NS = 2, 16
_NW = _NC * _NS
_GCH = 8
_CCH = 8


def _gate_body(x_ref, gw_ref, a0_ref, a1_ref, p0_ref, p1_ref):
    xb = x_ref[...]
    gw = gw_ref[...]
    scores = lax.dot_general(xb, gw, (((1,), (1,)), ((), ())),
                             preferred_element_type=jnp.float32)
    ii = lax.broadcasted_iota(jnp.int32, scores.shape, 1)
    m0 = jnp.max(scores, axis=1, keepdims=True)
    a0 = jnp.min(jnp.where(scores == m0, ii, _E), axis=1, keepdims=True)
    masked = jnp.where(ii == a0, -jnp.inf, scores)
    m1 = jnp.max(masked, axis=1, keepdims=True)
    a1 = jnp.min(jnp.where(masked == m1, ii, _E), axis=1, keepdims=True)
    t = jnp.exp(m1 - m0)
    s = 1.0 / (1.0 + t)
    a0_ref[...] = a0
    a1_ref[...] = a1
    p0_ref[...] = s
    p1_ref[...] = t * s


def _gate(x2, gate_W):
    return pl.pallas_call(
        _gate_body,
        grid=(_N // _TG,),
        in_specs=[
            pl.BlockSpec((_TG, _D), lambda i: (i, 0)),
            pl.BlockSpec((_E, _D), lambda i: (0, 0)),
        ],
        out_specs=[
            pl.BlockSpec((_TG, 1), lambda i: (i, 0)),
            pl.BlockSpec((_TG, 1), lambda i: (i, 0)),
            pl.BlockSpec((_TG, 1), lambda i: (i, 0)),
            pl.BlockSpec((_TG, 1), lambda i: (i, 0)),
        ],
        out_shape=[
            jax.ShapeDtypeStruct((_N, 1), jnp.int32),
            jax.ShapeDtypeStruct((_N, 1), jnp.int32),
            jax.ShapeDtypeStruct((_N, 1), jnp.float32),
            jax.ShapeDtypeStruct((_N, 1), jnp.float32),
        ],
    )(x2, gate_W)


def _sc_mesh():
    return plsc.VectorSubcoreMesh(core_axis_name="c", subcore_axis_name="s")


def _sc_wid():
    return lax.axis_index("s") * _NC + lax.axis_index("c")


def _gather_body(x_hbm, rt_hbm, out_hbm, idx_v,
                 a0, a1, b0, b1, sa0, sa1, sb0, sb1):
    wid = _sc_wid()
    rows_per_w = _RP // _NW
    npair = rows_per_w // (2 * _GCH)
    base = wid * rows_per_w
    pltpu.sync_copy(rt_hbm.at[pl.ds(base, rows_per_w)], idx_v)
    abufs = (a0, a1)
    bbufs = (b0, b1)
    sabufs = (sa0, sa1)
    sbbufs = (sb0, sb1)

    def issue(p):
        off = p * 2 * _GCH
        return (pltpu.async_copy(
                    x_hbm.at[idx_v.at[pl.ds(off, _GCH)]],
                    abufs[p % 2], sabufs[p % 2]),
                pltpu.async_copy(
                    x_hbm.at[idx_v.at[pl.ds(off + _GCH, _GCH)]],
                    bbufs[p % 2], sbbufs[p % 2]))

    cps = [issue(0)]
    for p in range(npair):
        if p + 1 < npair:
            cps.append(issue(p + 1))
        cpa, cpb = cps[p]
        cpa.wait()
        cpb.wait()
        off = base + p * 2 * _GCH
        pltpu.sync_copy(abufs[p % 2], out_hbm.at[pl.ds(off, _GCH)])
        pltpu.sync_copy(bbufs[p % 2], out_hbm.at[pl.ds(off + _GCH, _GCH)])


def _sc_gather(x2, row_token):
    k = functools.partial(
        pl.kernel,
        out_type=jax.ShapeDtypeStruct((_RP, _D), jnp.float32),
        mesh=_sc_mesh(),
        scratch_types=[
            pltpu.VMEM((_RP // _NW,), jnp.int32),
            pltpu.VMEM((_GCH, _D), jnp.float32),
            pltpu.VMEM((_GCH, _D), jnp.float32),
            pltpu.VMEM((_GCH, _D), jnp.float32),
            pltpu.VMEM((_GCH, _D), jnp.float32),
            pltpu.SemaphoreType.DMA,
            pltpu.SemaphoreType.DMA,
            pltpu.SemaphoreType.DMA,
            pltpu.SemaphoreType.DMA,
        ],
    )(_gather_body)
    return k(x2, row_token)


def _combine_body(yg_hbm, i0_hbm, i1_hbm, y_hbm, i0_v, i1_v,
                  ra0, ra1, rb0, rb1, sa0, sa1, sb0, sb1):
    wid = _sc_wid()
    rows_per_w = _N // _NW
    nch = rows_per_w // _CCH
    base = wid * rows_per_w
    pltpu.sync_copy(i0_hbm.at[pl.ds(base, rows_per_w)], i0_v)
    pltpu.sync_copy(i1_hbm.at[pl.ds(base, rows_per_w)], i1_v)
    rabufs = (ra0, ra1)
    rbbufs = (rb0, rb1)
    sabufs = (sa0, sa1)
    sbbufs = (sb0, sb1)

    def issue(c):
        sl = pl.ds(c * _CCH, _CCH)
        return (pltpu.async_copy(yg_hbm.at[i0_v.at[sl]], rabufs[c % 2],
                                 sabufs[c % 2]),
                pltpu.async_copy(yg_hbm.at[i1_v.at[sl]], rbbufs[c % 2],
                                 sbbufs[c % 2]))

    cps = [issue(0)]
    for c in range(nch):
        if c + 1 < nch:
            cps.append(issue(c + 1))
        cpa, cpb = cps[c]
        cpa.wait()
        cpb.wait()
        ra = rabufs[c % 2]
        rb = rbbufs[c % 2]

        def col(j, carry):
            sl = pl.ds(j * 16, 16)
            for r in range(_CCH):
                ra[r, sl] = ra[r, sl] + rb[r, sl]
            return carry

        lax.fori_loop(0, _D // 16, col, 0)
        pltpu.sync_copy(ra, y_hbm.at[pl.ds(base + c * _CCH, _CCH)])


def _sc_combine(yg, inv0, inv1):
    k = functools.partial(
        pl.kernel,
        out_type=jax.ShapeDtypeStruct((_N, _D), jnp.float32),
        mesh=_sc_mesh(),
        scratch_types=[
            pltpu.VMEM((_N // _NW,), jnp.int32),
            pltpu.VMEM((_N // _NW,), jnp.int32),
            pltpu.VMEM((_CCH, _D), jnp.float32),
            pltpu.VMEM((_CCH, _D), jnp.float32),
            pltpu.VMEM((_CCH, _D), jnp.float32),
            pltpu.VMEM((_CCH, _D), jnp.float32),
            pltpu.SemaphoreType.DMA,
            pltpu.SemaphoreType.DMA,
            pltpu.SemaphoreType.DMA,
            pltpu.SemaphoreType.DMA,
        ],
    )(_combine_body)
    return k(yg, inv0, inv1)


def _mlp_body(te_ref, tw_ref, x_ref, w1_ref, w2_ref, w3_ref, rw_ref,
              out_ref, acc_ref):
    j = pl.program_id(1)

    @pl.when(tw_ref[pl.program_id(0)] == 1)
    def _():
        xb = x_ref[...].astype(jnp.bfloat16)
        h1 = lax.dot_general(xb, w1_ref[0], (((1,), (1,)), ((), ())),
                             preferred_element_type=jnp.float32)
        h2 = lax.dot_general(xb, w2_ref[0], (((1,), (1,)), ((), ())),
                             preferred_element_type=jnp.float32)
        g = h1 * (1.0 / (1.0 + jnp.exp(-h1))) * h2
        part = lax.dot_general(g, w3_ref[0], (((1,), (1,)), ((), ())),
                               preferred_element_type=jnp.float32)

        @pl.when(j == 0)
        def _():
            acc_ref[...] = part

        @pl.when(j > 0)
        def _():
            acc_ref[...] += part

        @pl.when(j == _NH - 1)
        def _():
            out_ref[...] = acc_ref[...] * rw_ref[...]


def _grouped_mlp(xg, W1b, W2b, W3b, row_w, te, tw):
    grid_spec = pltpu.PrefetchScalarGridSpec(
        num_scalar_prefetch=2,
        grid=(_NT, _NH),
        in_specs=[
            pl.BlockSpec((_BR, _D), lambda i, j, te, tw: (i, 0)),
            pl.BlockSpec((1, _BH, _D), lambda i, j, te, tw: (te[i], j, 0)),
            pl.BlockSpec((1, _BH, _D), lambda i, j, te, tw: (te[i], j, 0)),
            pl.BlockSpec((1, _D, _BH), lambda i, j, te, tw: (te[i], 0, j)),
            pl.BlockSpec((_BR, 1), lambda i, j, te, tw: (i, 0)),
        ],
        out_specs=pl.BlockSpec((_BR, _D), lambda i, j, te, tw: (i, 0)),
        scratch_shapes=[pltpu.VMEM((_BR, _D), jnp.float32)],
    )
    return pl.pallas_call(
        _mlp_body,
        grid_spec=grid_spec,
        out_shape=jax.ShapeDtypeStruct((_RP, _D), jnp.float32),
        compiler_params=pltpu.CompilerParams(
            vmem_limit_bytes=63 * 1024 * 1024),
    )(te, tw, xg, W1b, W2b, W3b, row_w)


def kernel(x, gate_W, W1, W2, W3):
    x2 = x.reshape(_N, _D)

    a0, a1, p0, p1 = _gate(x2, gate_W)

    eflat = jnp.concatenate([a0, a1], axis=1).reshape(_P)
    wflat = jnp.concatenate([p0, p1], axis=1).reshape(_P)
    order = jnp.argsort(eflat).astype(jnp.int32)
    sorted_e = eflat[order]
    counts = jnp.bincount(eflat, length=_E).astype(jnp.int32)
    padded = ((counts + _BR - 1) // _BR) * _BR
    ends_p = jnp.cumsum(padded).astype(jnp.int32)
    starts_p = ends_p - padded
    starts_u = (jnp.cumsum(counts) - counts).astype(jnp.int32)
    jj = jnp.arange(_P, dtype=jnp.int32)
    dest = starts_p[sorted_e] + (jj - starts_u[sorted_e])
    row_token = jnp.zeros((_RP,), jnp.int32).at[dest].set(
        (order // _K).astype(jnp.int32))
    row_w = jnp.zeros((_RP,), jnp.float32).at[dest].set(
        wflat[order]).reshape(_RP, 1)
    pos = jnp.zeros((_P,), jnp.int32).at[order].set(dest)
    inv = pos.reshape(_N, _K)
    inv0 = inv[:, 0]
    inv1 = inv[:, 1]

    tile_base = jnp.arange(_NT, dtype=jnp.int32) * _BR
    te = jnp.minimum(
        jnp.searchsorted(ends_p, tile_base, side="right"), _E - 1
    ).astype(jnp.int32)
    tw = (tile_base < starts_p[te] + counts[te]).astype(jnp.int32)

    W1b = W1.astype(jnp.bfloat16)
    W2b = W2.astype(jnp.bfloat16)

    xg = _sc_gather(x2, row_token)
    yg = _grouped_mlp(xg, W1b, W2b, W3, row_w, te, tw)
    y2 = _sc_combine(yg, inv0, inv1)
    return y2.reshape(_B, _T, _D)


# speedup vs baseline: 1.0899x; 1.0899x over previous
import functools

import jax
import jax.numpy as jnp
from jax import lax
from jax.experimental import pallas as pl
from jax.experimental.pallas import tpu as pltpu
from jax.experimental.pallas import tpu_sc as plsc

_B, _T, _D, _H, _E, _K = 2, 2048, 2048, 2048, 8, 2
_N = _B * _T
_P = _N * _K
_BR = 512
_BH = 512
_RP = _P + _E * _BR - ((_P + _E * _BR) % _BR)
_NT = _RP // _BR
_NH = _H // _BH
_TG = 512

_NC, _NS = 2, 16
_NW = _NC * _NS
_GCH = 8
_CCH = 8


def _gate_body(x_ref, gw_ref, a0_ref, a1_ref, p0_ref, p1_ref):
    xb = x_ref[...]
    gw = gw_ref[...]
    scores = lax.dot_general(xb, gw, (((1,), (1,)), ((), ())),
                             preferred_element_type=jnp.float32)
    ii = lax.broadcasted_iota(jnp.int32, scores.shape, 1)
    m0 = jnp.max(scores, axis=1, keepdims=True)
    a0 = jnp.min(jnp.where(scores == m0, ii, _E), axis=1, keepdims=True)
    masked = jnp.where(ii == a0, -jnp.inf, scores)
    m1 = jnp.max(masked, axis=1, keepdims=True)
    a1 = jnp.min(jnp.where(masked == m1, ii, _E), axis=1, keepdims=True)
    t = jnp.exp(m1 - m0)
    s = 1.0 / (1.0 + t)
    a0_ref[...] = a0
    a1_ref[...] = a1
    p0_ref[...] = s
    p1_ref[...] = t * s


def _gate(x2, gate_W):
    return pl.pallas_call(
        _gate_body,
        grid=(_N // _TG,),
        in_specs=[
            pl.BlockSpec((_TG, _D), lambda i: (i, 0)),
            pl.BlockSpec((_E, _D), lambda i: (0, 0)),
        ],
        out_specs=[
            pl.BlockSpec((_TG, 1), lambda i: (i, 0)),
            pl.BlockSpec((_TG, 1), lambda i: (i, 0)),
            pl.BlockSpec((_TG, 1), lambda i: (i, 0)),
            pl.BlockSpec((_TG, 1), lambda i: (i, 0)),
        ],
        out_shape=[
            jax.ShapeDtypeStruct((_N, 1), jnp.int32),
            jax.ShapeDtypeStruct((_N, 1), jnp.int32),
            jax.ShapeDtypeStruct((_N, 1), jnp.float32),
            jax.ShapeDtypeStruct((_N, 1), jnp.float32),
        ],
    )(x2, gate_W)


def _sc_mesh():
    return plsc.VectorSubcoreMesh(core_axis_name="c", subcore_axis_name="s")


def _sc_wid():
    return lax.axis_index("s") * _NC + lax.axis_index("c")


def _gather_body(x_hbm, rt_hbm, out_hbm, idx_v,
                 a0, a1, b0, b1, sa0, sa1, sb0, sb1):
    wid = _sc_wid()
    rows_per_w = _RP // _NW
    npair = rows_per_w // (2 * _GCH)
    base = wid * rows_per_w
    pltpu.sync_copy(rt_hbm.at[pl.ds(base, rows_per_w)], idx_v)
    abufs = (a0, a1)
    bbufs = (b0, b1)
    sabufs = (sa0, sa1)
    sbbufs = (sb0, sb1)

    def issue(p):
        off = p * 2 * _GCH
        return (pltpu.async_copy(
                    x_hbm.at[idx_v.at[pl.ds(off, _GCH)]],
                    abufs[p % 2], sabufs[p % 2]),
                pltpu.async_copy(
                    x_hbm.at[idx_v.at[pl.ds(off + _GCH, _GCH)]],
                    bbufs[p % 2], sbbufs[p % 2]))

    cps = [issue(0)]
    for p in range(npair):
        if p + 1 < npair:
            cps.append(issue(p + 1))
        cpa, cpb = cps[p]
        cpa.wait()
        cpb.wait()
        off = base + p * 2 * _GCH
        pltpu.sync_copy(abufs[p % 2], out_hbm.at[pl.ds(off, _GCH)])
        pltpu.sync_copy(bbufs[p % 2], out_hbm.at[pl.ds(off + _GCH, _GCH)])


def _sc_gather(x2, row_token):
    k = functools.partial(
        pl.kernel,
        out_type=jax.ShapeDtypeStruct((_RP, _D), jnp.float32),
        mesh=_sc_mesh(),
        scratch_types=[
            pltpu.VMEM((_RP // _NW,), jnp.int32),
            pltpu.VMEM((_GCH, _D), jnp.float32),
            pltpu.VMEM((_GCH, _D), jnp.float32),
            pltpu.VMEM((_GCH, _D), jnp.float32),
            pltpu.VMEM((_GCH, _D), jnp.float32),
            pltpu.SemaphoreType.DMA,
            pltpu.SemaphoreType.DMA,
            pltpu.SemaphoreType.DMA,
            pltpu.SemaphoreType.DMA,
        ],
    )(_gather_body)
    return k(x2, row_token)


def _combine_body(yg_hbm, i0_hbm, i1_hbm, y_hbm, i0_v, i1_v,
                  ra0, ra1, rb0, rb1, sa0, sa1, sb0, sb1):
    wid = _sc_wid()
    rows_per_w = _N // _NW
    nch = rows_per_w // _CCH
    base = wid * rows_per_w
    pltpu.sync_copy(i0_hbm.at[pl.ds(base, rows_per_w)], i0_v)
    pltpu.sync_copy(i1_hbm.at[pl.ds(base, rows_per_w)], i1_v)
    rabufs = (ra0, ra1)
    rbbufs = (rb0, rb1)
    sabufs = (sa0, sa1)
    sbbufs = (sb0, sb1)

    def issue(c):
        sl = pl.ds(c * _CCH, _CCH)
        return (pltpu.async_copy(yg_hbm.at[i0_v.at[sl]], rabufs[c % 2],
                                 sabufs[c % 2]),
                pltpu.async_copy(yg_hbm.at[i1_v.at[sl]], rbbufs[c % 2],
                                 sbbufs[c % 2]))

    cps = [issue(0)]
    for c in range(nch):
        if c + 1 < nch:
            cps.append(issue(c + 1))
        cpa, cpb = cps[c]
        cpa.wait()
        cpb.wait()
        ra = rabufs[c % 2]
        rb = rbbufs[c % 2]

        def col(j, carry):
            sl = pl.ds(j * 16, 16)
            for r in range(_CCH):
                ra[r, sl] = ra[r, sl] + rb[r, sl]
            return carry

        lax.fori_loop(0, _D // 16, col, 0)
        pltpu.sync_copy(ra, y_hbm.at[pl.ds(base + c * _CCH, _CCH)])


def _sc_combine(yg, inv0, inv1):
    k = functools.partial(
        pl.kernel,
        out_type=jax.ShapeDtypeStruct((_N, _D), jnp.float32),
        mesh=_sc_mesh(),
        scratch_types=[
            pltpu.VMEM((_N // _NW,), jnp.int32),
            pltpu.VMEM((_N // _NW,), jnp.int32),
            pltpu.VMEM((_CCH, _D), jnp.float32),
            pltpu.VMEM((_CCH, _D), jnp.float32),
            pltpu.VMEM((_CCH, _D), jnp.float32),
            pltpu.VMEM((_CCH, _D), jnp.float32),
            pltpu.SemaphoreType.DMA,
            pltpu.SemaphoreType.DMA,
            pltpu.SemaphoreType.DMA,
            pltpu.SemaphoreType.DMA,
        ],
    )(_combine_body)
    return k(yg, inv0, inv1)


def _mlp_body(te_ref, tw_ref, x_ref, w1_ref, w2_ref, w3_ref, rw_ref,
              out_ref, acc_ref):
    j = pl.program_id(1)

    @pl.when(tw_ref[pl.program_id(0)] == 1)
    def _():
        xb = x_ref[...]
        h1 = lax.dot_general(xb, w1_ref[0], (((1,), (1,)), ((), ())),
                             preferred_element_type=jnp.float32)
        h2 = lax.dot_general(xb, w2_ref[0], (((1,), (1,)), ((), ())),
                             preferred_element_type=jnp.float32)
        g = h1 * (1.0 / (1.0 + jnp.exp(-h1))) * h2
        part = lax.dot_general(g, w3_ref[0], (((1,), (1,)), ((), ())),
                               preferred_element_type=jnp.float32)

        @pl.when(j == 0)
        def _():
            acc_ref[...] = part

        @pl.when(j > 0)
        def _():
            acc_ref[...] += part

        @pl.when(j == _NH - 1)
        def _():
            out_ref[...] = acc_ref[...] * rw_ref[...]


def _grouped_mlp(xg, W1b, W2b, W3b, row_w, te, tw):
    grid_spec = pltpu.PrefetchScalarGridSpec(
        num_scalar_prefetch=2,
        grid=(_NT, _NH),
        in_specs=[
            pl.BlockSpec((_BR, _D), lambda i, j, te, tw: (i, 0)),
            pl.BlockSpec((1, _BH, _D), lambda i, j, te, tw: (te[i], j, 0)),
            pl.BlockSpec((1, _BH, _D), lambda i, j, te, tw: (te[i], j, 0)),
            pl.BlockSpec((1, _D, _BH), lambda i, j, te, tw: (te[i], 0, j)),
            pl.BlockSpec((_BR, 1), lambda i, j, te, tw: (i, 0)),
        ],
        out_specs=pl.BlockSpec((_BR, _D), lambda i, j, te, tw: (i, 0)),
        scratch_shapes=[pltpu.VMEM((_BR, _D), jnp.float32)],
    )
    return pl.pallas_call(
        _mlp_body,
        grid_spec=grid_spec,
        out_shape=jax.ShapeDtypeStruct((_RP, _D), jnp.float32),
        compiler_params=pltpu.CompilerParams(
            vmem_limit_bytes=63 * 1024 * 1024),
    )(te, tw, xg, W1b, W2b, W3b, row_w)


def kernel(x, gate_W, W1, W2, W3):
    x2 = x.reshape(_N, _D)

    a0, a1, p0, p1 = _gate(x2, gate_W)

    eflat = jnp.concatenate([a0, a1], axis=1).reshape(_P)
    wflat = jnp.concatenate([p0, p1], axis=1).reshape(_P)
    order = jnp.argsort(eflat).astype(jnp.int32)
    sorted_e = eflat[order]
    counts = jnp.bincount(eflat, length=_E).astype(jnp.int32)
    padded = ((counts + _BR - 1) // _BR) * _BR
    ends_p = jnp.cumsum(padded).astype(jnp.int32)
    starts_p = ends_p - padded
    starts_u = (jnp.cumsum(counts) - counts).astype(jnp.int32)
    jj = jnp.arange(_P, dtype=jnp.int32)
    dest = starts_p[sorted_e] + (jj - starts_u[sorted_e])
    row_token = jnp.zeros((_RP,), jnp.int32).at[dest].set(
        (order // _K).astype(jnp.int32))
    row_w = jnp.zeros((_RP,), jnp.float32).at[dest].set(
        wflat[order]).reshape(_RP, 1)
    pos = jnp.zeros((_P,), jnp.int32).at[order].set(dest)
    inv = pos.reshape(_N, _K)
    inv0 = inv[:, 0]
    inv1 = inv[:, 1]

    tile_base = jnp.arange(_NT, dtype=jnp.int32) * _BR
    te = jnp.minimum(
        jnp.searchsorted(ends_p, tile_base, side="right"), _E - 1
    ).astype(jnp.int32)
    tw = (tile_base < starts_p[te] + counts[te]).astype(jnp.int32)

    xg = _sc_gather(x2, row_token)
    yg = _grouped_mlp(xg, W1, W2, W3, row_w, te, tw)
    y2 = _sc_combine(yg, inv0, inv1)
    return y2.reshape(_B, _T, _D)
